# Initial kernel scaffold; baseline (speedup 1.0000x reference)
import jax
import jax.numpy as jnp
from jax.experimental import pallas as pl


def kernel(x, embed):
    raise NotImplementedError("write your pallas kernel here")


# speedup vs baseline: 1.1733x; 1.1733x over previous
import functools

import jax
import jax.numpy as jnp
from jax import lax
from jax.experimental import pallas as pl
from jax.experimental.pallas import tpu as pltpu
from jax.experimental.pallas import tpu_sc as plsc

DIM_ = 32
K_ = 8192
M_ = 16384
BLK_ = 256

NC_ = 2
NS_ = 16
NW_ = NC_ * NS_
BPW_ = M_ // NW_


def _argmin_body(x_ref, et_ref, ind_ref):
    x = x_ref[...]
    et = et_ref[...]
    xs = jnp.sum(x * x, axis=1, keepdims=True)
    es = jnp.sum(et * et, axis=0, keepdims=True)
    mm = jnp.dot(x, et, preferred_element_type=jnp.float32)
    dist = -((xs - 2.0 * mm) + es)
    nchunk = 2
    cw = K_ // nchunk
    acc_v = jnp.full((BLK_, 1), -jnp.inf, jnp.float32)
    acc_i = jnp.zeros((BLK_, 1), jnp.int32)
    for c in range(nchunk):
        ch = dist[:, c * cw:(c + 1) * cw]
        cmax = jnp.max(ch, axis=1, keepdims=True)
        cidx = jnp.argmax(ch, axis=1, keepdims=True).astype(jnp.int32) + c * cw
        keep = (acc_v > cmax) | ((acc_v == cmax) & (acc_i < cidx))
        acc_v = jnp.where(keep, acc_v, cmax).astype(jnp.bfloat16).astype(jnp.float32)
        acc_i = jnp.where(keep, acc_i, cidx)
    ind_ref[...] = acc_i.reshape(1, 1, BLK_)


def _argmin_indices(flat, et):
    grid = (M_ // BLK_,)
    out = pl.pallas_call(
        _argmin_body,
        grid=grid,
        in_specs=[
            pl.BlockSpec((BLK_, DIM_), lambda i: (i, 0)),
            pl.BlockSpec((DIM_, K_), lambda i: (0, 0)),
        ],
        out_specs=pl.BlockSpec((1, 1, BLK_), lambda i: (i, 0, 0)),
        out_shape=jax.ShapeDtypeStruct((M_ // BLK_, 1, BLK_), jnp.int32),
    )(flat, et)
    return out.reshape(M_)


@functools.cache
def _make_gather_rows():
    @functools.partial(
        pl.kernel,
        out_type=jax.ShapeDtypeStruct((M_, DIM_), jnp.float32),
        mesh=plsc.VectorSubcoreMesh(
            core_axis_name="c", subcore_axis_name="s",
            num_cores=NC_, num_subcores=NS_,
        ),
        scratch_types=[
            pltpu.VMEM((BPW_,), jnp.int32),
            pltpu.VMEM((BPW_, DIM_), jnp.float32),
            pltpu.SemaphoreType.DMA,
        ],
        compiler_params=pltpu.CompilerParams(use_tc_tiling_on_sc=False),
    )
    def _gather_rows(table_hbm, idx_hbm, out_hbm, idx_v, rows_v, sem):
        wid = lax.axis_index("s") * NC_ + lax.axis_index("c")
        base = wid * BPW_
        pltpu.sync_copy(idx_hbm.at[pl.ds(base, BPW_)], idx_v)
        pltpu.async_copy(table_hbm.at[idx_v], rows_v, sem).wait()
        pltpu.sync_copy(rows_v, out_hbm.at[pl.ds(base, BPW_)])

    return _gather_rows


def kernel(x, embed):
    shape = x.shape
    flat = x.reshape(-1, shape[-1])
    et = embed.T
    ind = _argmin_indices(flat, et)
    quantize = _make_gather_rows()(embed, ind)
    return quantize.reshape(shape), ind.reshape(shape[:-1])


# speedup vs baseline: 1.2565x; 1.0709x over previous
import functools

import jax
import jax.numpy as jnp
from jax import lax
from jax.experimental import pallas as pl
from jax.experimental.pallas import tpu as pltpu
from jax.experimental.pallas import tpu_sc as plsc

DIM_ = 32
K_ = 8192
M_ = 16384
BLK_ = 512

NC_ = 2
NS_ = 16
NW_ = NC_ * NS_
BPW_ = M_ // NW_


def _argmin_body(x_ref, et_ref, ind_ref):
    x = x_ref[...]
    et = et_ref[...]
    xs = jnp.sum(x * x, axis=1, keepdims=True)
    es = jnp.sum(et * et, axis=0, keepdims=True)
    mm = jnp.dot(x, et, preferred_element_type=jnp.float32)
    dist = -((xs - 2.0 * mm) + es)
    nchunk = 2
    cw = K_ // nchunk
    acc_v = jnp.full((BLK_, 1), -jnp.inf, jnp.float32)
    acc_i = jnp.zeros((BLK_, 1), jnp.int32)
    for c in range(nchunk):
        ch = dist[:, c * cw:(c + 1) * cw]
        cmax = jnp.max(ch, axis=1, keepdims=True)
        cidx = jnp.argmax(ch, axis=1, keepdims=True).astype(jnp.int32) + c * cw
        keep = (acc_v > cmax) | ((acc_v == cmax) & (acc_i < cidx))
        acc_v = jnp.where(keep, acc_v, cmax).astype(jnp.bfloat16).astype(jnp.float32)
        acc_i = jnp.where(keep, acc_i, cidx)
    ind_ref[...] = acc_i.reshape(1, 1, BLK_)


def _argmin_indices(flat, et):
    grid = (M_ // BLK_,)
    out = pl.pallas_call(
        _argmin_body,
        grid=grid,
        in_specs=[
            pl.BlockSpec((BLK_, DIM_), lambda i: (i, 0)),
            pl.BlockSpec((DIM_, K_), lambda i: (0, 0)),
        ],
        out_specs=pl.BlockSpec((1, 1, BLK_), lambda i: (i, 0, 0)),
        out_shape=jax.ShapeDtypeStruct((M_ // BLK_, 1, BLK_), jnp.int32),
    )(flat, et)
    return out.reshape(M_)


@functools.cache
def _make_gather_rows():
    @functools.partial(
        pl.kernel,
        out_type=jax.ShapeDtypeStruct((M_, DIM_), jnp.float32),
        mesh=plsc.VectorSubcoreMesh(
            core_axis_name="c", subcore_axis_name="s",
            num_cores=NC_, num_subcores=NS_,
        ),
        scratch_types=[
            pltpu.VMEM((BPW_,), jnp.int32),
            pltpu.VMEM((BPW_, DIM_), jnp.float32),
            pltpu.SemaphoreType.DMA,
        ],
        compiler_params=pltpu.CompilerParams(use_tc_tiling_on_sc=False),
    )
    def _gather_rows(table_hbm, idx_hbm, out_hbm, idx_v, rows_v, sem):
        wid = lax.axis_index("s") * NC_ + lax.axis_index("c")
        base = wid * BPW_
        pltpu.sync_copy(idx_hbm.at[pl.ds(base, BPW_)], idx_v)
        pltpu.async_copy(table_hbm.at[idx_v], rows_v, sem).wait()
        pltpu.sync_copy(rows_v, out_hbm.at[pl.ds(base, BPW_)])

    return _gather_rows


def kernel(x, embed):
    shape = x.shape
    flat = x.reshape(-1, shape[-1])
    et = embed.T
    ind = _argmin_indices(flat, et)
    quantize = _make_gather_rows()(embed, ind)
    return quantize.reshape(shape), ind.reshape(shape[:-1])


# speedup vs baseline: 1.3514x; 1.0755x over previous
import functools

import jax
import jax.numpy as jnp
from jax import lax
from jax.experimental import pallas as pl
from jax.experimental.pallas import tpu as pltpu
from jax.experimental.pallas import tpu_sc as plsc

DIM_ = 32
K_ = 8192
M_ = 16384
BLK_ = 512

NC_ = 2
NS_ = 16
NW_ = NC_ * NS_
BPW_ = M_ // NW_


def _argmin_body(x_ref, et_ref, ind_ref):
    x = x_ref[...]
    et = et_ref[...]
    xs = jnp.sum(x * x, axis=1, keepdims=True)
    es = jnp.sum(et * et, axis=0, keepdims=True)
    mm = jnp.dot(x, et, preferred_element_type=jnp.float32)
    ndist = (xs - 2.0 * mm) + es
    nchunk = 2
    cw = K_ // nchunk
    acc_v = jnp.full((BLK_, 1), jnp.inf, jnp.float32)
    acc_i = jnp.zeros((BLK_, 1), jnp.int32)
    for c in range(nchunk):
        ch = ndist[:, c * cw:(c + 1) * cw]
        cmin = jnp.min(ch, axis=1, keepdims=True)
        cidx = jnp.argmin(ch, axis=1, keepdims=True).astype(jnp.int32) + c * cw
        keep = (acc_v < cmin) | ((acc_v == cmin) & (acc_i < cidx))
        acc_v = jnp.where(keep, acc_v, cmin).astype(jnp.bfloat16).astype(jnp.float32)
        acc_i = jnp.where(keep, acc_i, cidx)
    ind_ref[...] = acc_i.reshape(1, 1, BLK_)


def _argmin_indices(flat, et):
    grid = (M_ // BLK_,)
    out = pl.pallas_call(
        _argmin_body,
        grid=grid,
        in_specs=[
            pl.BlockSpec((BLK_, DIM_), lambda i: (i, 0)),
            pl.BlockSpec((DIM_, K_), lambda i: (0, 0)),
        ],
        out_specs=pl.BlockSpec((1, 1, BLK_), lambda i: (i, 0, 0)),
        out_shape=jax.ShapeDtypeStruct((M_ // BLK_, 1, BLK_), jnp.int32),
    )(flat, et)
    return out.reshape(M_)


@functools.cache
def _make_gather_rows():
    @functools.partial(
        pl.kernel,
        out_type=jax.ShapeDtypeStruct((M_, DIM_), jnp.float32),
        mesh=plsc.VectorSubcoreMesh(
            core_axis_name="c", subcore_axis_name="s",
            num_cores=NC_, num_subcores=NS_,
        ),
        scratch_types=[
            pltpu.VMEM((BPW_,), jnp.int32),
            pltpu.VMEM((BPW_, DIM_), jnp.float32),
            pltpu.SemaphoreType.DMA,
        ],
        compiler_params=pltpu.CompilerParams(use_tc_tiling_on_sc=False),
    )
    def _gather_rows(table_hbm, idx_hbm, out_hbm, idx_v, rows_v, sem):
        wid = lax.axis_index("s") * NC_ + lax.axis_index("c")
        base = wid * BPW_
        pltpu.sync_copy(idx_hbm.at[pl.ds(base, BPW_)], idx_v)
        pltpu.async_copy(table_hbm.at[idx_v], rows_v, sem).wait()
        pltpu.sync_copy(rows_v, out_hbm.at[pl.ds(base, BPW_)])

    return _gather_rows


def kernel(x, embed):
    shape = x.shape
    flat = x.reshape(-1, shape[-1])
    et = embed.T
    ind = _argmin_indices(flat, et)
    quantize = _make_gather_rows()(embed, ind)
    return quantize.reshape(shape), ind.reshape(shape[:-1])
